# baseline (speedup 1.0000x reference)
import functools

import jax
import jax.numpy as jnp
from jax import lax
from jax.experimental import pallas as pl
from jax.experimental.pallas import tpu as pltpu
from jax.experimental.pallas import tpu_sc as plsc

NC = 2
NS = 16
NW = NC * NS

CH = 80
ITERS = 128
NP = 10240
BLK = 1000


def _seg_sum_sc(h, eidx, with_deg):
  n, d = h.shape
  rpt = NP // NS

  half = ITERS // 2
  out_type = [jax.ShapeDtypeStruct((NC, NP, d), jnp.float32)]
  scratch = [
      pltpu.VMEM((half, CH), jnp.int32),
      pltpu.VMEM((half, CH), jnp.int32),
      pltpu.VMEM((2, CH, d), jnp.float32),
      pltpu.VMEM_SHARED((NP, d), jnp.float32),
      pltpu.SemaphoreType.DMA,
      pltpu.SemaphoreType.DMA,
  ]
  if with_deg:
    out_type.append(jax.ShapeDtypeStruct((NC, NP), jnp.float32))
    scratch += [
        pltpu.VMEM((CH,), jnp.float32),
        pltpu.VMEM_SHARED((NP,), jnp.float32),
    ]

  mesh = plsc.VectorSubcoreMesh(core_axis_name="c", subcore_axis_name="s",
                                num_cores=NC, num_subcores=NS)

  def body(h_hbm, src_hbm, dst_hbm, z2_hbm, z1_hbm, *refs):
    if with_deg:
      out_hbm, deg_hbm, src_v, dst_v, rows_v, acc, gs0, gs1, ones_v, dacc = refs
    else:
      out_hbm, src_v, dst_v, rows_v, acc, gs0, gs1 = refs
    gsems = (gs0, gs1)
    c = lax.axis_index("c")
    s = lax.axis_index("s")
    wid = c * NS + s

    pltpu.sync_copy(z2_hbm, acc.at[pl.ds(s * rpt, rpt)])

    if with_deg:
      one16 = jnp.ones((16,), jnp.float32)
      for jj in range(CH // 16):
        ones_v[pl.ds(jj * 16, 16)] = one16

      @pl.when(s == 0)
      def _():
        pltpu.sync_copy(z1_hbm, dacc)

    plsc.subcore_barrier()

    for hi in range(2):
      h0, hn = hi * half, half
      pltpu.sync_copy(src_hbm.at[wid, pl.ds(h0, hn)], src_v)
      pltpu.sync_copy(dst_hbm.at[wid, pl.ds(h0, hn)], dst_v)
      descs = {0: pltpu.async_copy(h_hbm.at[src_v.at[0]], rows_v.at[0],
                                   gsems[0])}
      for j in range(hn):
        b = j % 2
        descs[j].wait()
        if j + 1 < hn:
          b2 = (j + 1) % 2
          descs[j + 1] = pltpu.async_copy(h_hbm.at[src_v.at[j + 1]],
                                          rows_v.at[b2], gsems[b2])
        pltpu.sync_copy(rows_v.at[b], acc.at[dst_v.at[j]], add=True)
        if with_deg:
          pltpu.sync_copy(ones_v, dacc.at[dst_v.at[j]], add=True)

    plsc.subcore_barrier()

    pltpu.sync_copy(acc.at[pl.ds(s * rpt, rpt)],
                    out_hbm.at[c, pl.ds(s * rpt, rpt)])
    if with_deg:
      @pl.when(s == 0)
      def _():
        pltpu.sync_copy(dacc, deg_hbm.at[c])

  fn = pl.kernel(body, out_type=out_type, mesh=mesh, scratch_types=scratch)
  z2 = jnp.zeros((NP // NS, d), jnp.float32)
  z1 = jnp.zeros((NP,), jnp.float32)
  res = fn(h, eidx[0], eidx[1], z2, z1)
  return res if with_deg else res[0]


def _layer_body(h_ref, agg_ref, deg_ref, ws_ref, wn_ref, b_ref, out_ref):
  a = agg_ref[0] + agg_ref[1]
  dg = jnp.sum(deg_ref[...], axis=0)
  aggn = a / jnp.maximum(dg, 1.0)
  hs = lax.dot_general(h_ref[...], ws_ref[...], (((1,), (1,)), ((), ())),
                       preferred_element_type=jnp.float32)
  hn = lax.dot_general(aggn, wn_ref[...], (((1,), (1,)), ((), ())),
                       preferred_element_type=jnp.float32)
  out_ref[...] = jnp.maximum(hs + hn + b_ref[...], 0.0)


def _layer_tc(h, agg, deg, ws, wn, b):
  n, d = h.shape
  grid = (n // BLK,)
  return pl.pallas_call(
      _layer_body,
      grid=grid,
      in_specs=[
          pl.BlockSpec((BLK, d), lambda i: (i, 0)),
          pl.BlockSpec((NC, BLK, d), lambda i: (0, i, 0)),
          pl.BlockSpec((NC, BLK, 1), lambda i: (0, i, 0)),
          pl.BlockSpec((d, d), lambda i: (0, 0)),
          pl.BlockSpec((d, d), lambda i: (0, 0)),
          pl.BlockSpec((1, d), lambda i: (0, 0)),
      ],
      out_specs=pl.BlockSpec((BLK, d), lambda i: (i, 0)),
      out_shape=jax.ShapeDtypeStruct((n, d), jnp.float32),
  )(h, agg, deg, ws, wn, b)


def _heads_body(h_ref, agg_ref, deg_ref, ws_ref, wn_ref, b_ref,
                wmu_ref, wsig_ref, weps_ref, bmu_ref, bsig_ref, beps_ref,
                gidx_ref, wv_ref, bv_ref,
                adv_ref, val_ref, gsum_ref, gcnt_ref):
  i = pl.program_id(0)
  ni = pl.num_programs(0)

  a = agg_ref[0] + agg_ref[1]
  dg = jnp.sum(deg_ref[...], axis=0)
  aggn = a / jnp.maximum(dg, 1.0)
  hs = lax.dot_general(h_ref[...], ws_ref[...], (((1,), (1,)), ((), ())),
                       preferred_element_type=jnp.float32)
  hn = lax.dot_general(aggn, wn_ref[...], (((1,), (1,)), ((), ())),
                       preferred_element_type=jnp.float32)
  h2 = jnp.maximum(hs + hn + b_ref[...], 0.0)

  weff = wmu_ref[...] + wsig_ref[...] * weps_ref[...]
  beff = bmu_ref[...] + bsig_ref[...] * beps_ref[...]
  adv_ref[...] = lax.dot_general(h2, weff, (((1,), (1,)), ((), ())),
                                 preferred_element_type=jnp.float32) + beff

  gi = gidx_ref[0]
  gids = lax.broadcasted_iota(jnp.int32, (64, BLK), 0)
  onehot = (gids == jnp.broadcast_to(gi, (64, BLK))).astype(jnp.float32)

  @pl.when(i == 0)
  def _():
    gsum_ref[...] = jnp.zeros_like(gsum_ref)
    gcnt_ref[...] = jnp.zeros_like(gcnt_ref)

  gsum_ref[...] += lax.dot_general(onehot, h2, (((1,), (0,)), ((), ())),
                                   preferred_element_type=jnp.float32)
  gcnt_ref[...] += jnp.sum(onehot, axis=1, keepdims=True)

  @pl.when(i == ni - 1)
  def _():
    gmean = gsum_ref[...] / jnp.maximum(gcnt_ref[...], 1.0)
    vv = jnp.sum(gmean * wv_ref[...], axis=1, keepdims=True)
    val_ref[...] = vv + bv_ref[0, 0]


def _heads_tc(h, agg, deg, ws, wn, b, wmu, wsig, weps, bmu, bsig, beps,
              gidx, wv, bv):
  n, d = h.shape
  out_dim = wmu.shape[0]
  g = 64
  grid = (n // BLK,)
  return pl.pallas_call(
      _heads_body,
      grid=grid,
      in_specs=[
          pl.BlockSpec((BLK, d), lambda i: (i, 0)),
          pl.BlockSpec((NC, BLK, d), lambda i: (0, i, 0)),
          pl.BlockSpec((NC, BLK, 1), lambda i: (0, i, 0)),
          pl.BlockSpec((d, d), lambda i: (0, 0)),
          pl.BlockSpec((d, d), lambda i: (0, 0)),
          pl.BlockSpec((1, d), lambda i: (0, 0)),
          pl.BlockSpec((out_dim, d), lambda i: (0, 0)),
          pl.BlockSpec((out_dim, d), lambda i: (0, 0)),
          pl.BlockSpec((out_dim, d), lambda i: (0, 0)),
          pl.BlockSpec((1, out_dim), lambda i: (0, 0)),
          pl.BlockSpec((1, out_dim), lambda i: (0, 0)),
          pl.BlockSpec((1, out_dim), lambda i: (0, 0)),
          pl.BlockSpec((1, 1, BLK), lambda i: (i, 0, 0)),
          pl.BlockSpec((1, d), lambda i: (0, 0)),
          pl.BlockSpec((1, 1), lambda i: (0, 0)),
      ],
      out_specs=[
          pl.BlockSpec((BLK, out_dim), lambda i: (i, 0)),
          pl.BlockSpec((g, 1), lambda i: (0, 0)),
      ],
      out_shape=[
          jax.ShapeDtypeStruct((n, out_dim), jnp.float32),
          jax.ShapeDtypeStruct((g, 1), jnp.float32),
      ],
      scratch_shapes=[
          pltpu.VMEM((g, d), jnp.float32),
          pltpu.VMEM((g, 1), jnp.float32),
      ],
  )(h, agg, deg, ws, wn, b, wmu, wsig, weps, bmu, bsig, beps, gidx, wv, bv)


def kernel(x, edge_index, graph_indices, W1s, W1n, b1, W2s, W2n, b2,
           w_mu, w_sigma, w_eps, b_mu, b_sigma, b_eps, Wv, bv):
  n, d = x.shape
  e = edge_index.shape[1]
  ep = NW * ITERS * CH
  pad = jnp.stack([jnp.zeros((ep - e,), jnp.int32),
                   jnp.full((ep - e,), NP - 1, jnp.int32)])
  eip = jnp.concatenate([edge_index, pad], axis=1)
  eidx = (eip[0].reshape(NW, ITERS, CH), eip[1].reshape(NW, ITERS, CH))

  agg1, deg = _seg_sum_sc(x, eidx, with_deg=True)
  deg3 = deg.reshape(NC, NP, 1)
  h1 = _layer_tc(x, agg1, deg3, W1s, W1n, b1.reshape(1, d))

  agg2 = _seg_sum_sc(h1, eidx, with_deg=False)

  adv, val = _heads_tc(
      h1, agg2, deg3, W2s, W2n, b2.reshape(1, d),
      w_mu, w_sigma, w_eps,
      b_mu.reshape(1, -1), b_sigma.reshape(1, -1), b_eps.reshape(1, -1),
      graph_indices.reshape(n // BLK, 1, BLK), Wv, bv.reshape(1, 1))
  return adv, val


# speedup vs baseline: 2.3522x; 2.3522x over previous
import functools

import jax
import jax.numpy as jnp
from jax import lax
from jax.experimental import pallas as pl
from jax.experimental.pallas import tpu as pltpu
from jax.experimental.pallas import tpu_sc as plsc

NC = 2
NS = 16
NW = NC * NS

CH = 80
ITERS = 125
NP = 10240
BLK = 1000


def _seg_sum_sc(h, eidx, with_deg):
  n, d = h.shape
  rpt = NP // NS

  out_type = [jax.ShapeDtypeStruct((NC, NP, d), jnp.float32)]
  scratch = [
      pltpu.VMEM((ITERS, CH), jnp.int32),
      pltpu.VMEM((ITERS, CH), jnp.int32),
      pltpu.VMEM((CH, d), jnp.float32),
      pltpu.VMEM_SHARED((NP, d), jnp.float32),
  ]
  if with_deg:
    out_type.append(jax.ShapeDtypeStruct((NC, NP), jnp.float32))
    scratch += [
        pltpu.VMEM((CH,), jnp.float32),
        pltpu.VMEM_SHARED((NP,), jnp.float32),
    ]

  mesh = plsc.VectorSubcoreMesh(core_axis_name="c", subcore_axis_name="s",
                                num_cores=NC, num_subcores=NS)

  def body(h_hbm, src_hbm, dst_hbm, z2_hbm, z1_hbm, *refs):
    if with_deg:
      out_hbm, deg_hbm, src_v, dst_v, rows_v, acc, ones_v, dacc = refs
    else:
      out_hbm, src_v, dst_v, rows_v, acc = refs
    c = lax.axis_index("c")
    s = lax.axis_index("s")
    wid = c * NS + s

    pltpu.sync_copy(z2_hbm, acc.at[pl.ds(s * rpt, rpt)])

    if with_deg:
      one16 = jnp.ones((16,), jnp.float32)
      for jj in range(CH // 16):
        ones_v[pl.ds(jj * 16, 16)] = one16

      @pl.when(s == 0)
      def _():
        pltpu.sync_copy(z1_hbm, dacc)

    plsc.subcore_barrier()

    pltpu.sync_copy(src_hbm.at[wid], src_v)
    pltpu.sync_copy(dst_hbm.at[wid], dst_v)

    def step(j, carry):
      pltpu.sync_copy(h_hbm.at[src_v.at[j]], rows_v)
      pltpu.sync_copy(rows_v, acc.at[dst_v.at[j]], add=True)
      if with_deg:
        pltpu.sync_copy(ones_v, dacc.at[dst_v.at[j]], add=True)
      return carry
    lax.fori_loop(0, ITERS, step, 0)

    plsc.subcore_barrier()

    pltpu.sync_copy(acc.at[pl.ds(s * rpt, rpt)],
                    out_hbm.at[c, pl.ds(s * rpt, rpt)])
    if with_deg:
      @pl.when(s == 0)
      def _():
        pltpu.sync_copy(dacc, deg_hbm.at[c])

  fn = pl.kernel(body, out_type=out_type, mesh=mesh, scratch_types=scratch)
  z2 = jnp.zeros((NP // NS, d), jnp.float32)
  z1 = jnp.zeros((NP,), jnp.float32)
  res = fn(h, eidx[0], eidx[1], z2, z1)
  return res if with_deg else res[0]


def _layer_body(h_ref, agg_ref, deg_ref, ws_ref, wn_ref, b_ref, out_ref):
  a = agg_ref[0] + agg_ref[1]
  dg = jnp.sum(deg_ref[...], axis=0)
  aggn = a / jnp.maximum(dg, 1.0)
  hs = lax.dot_general(h_ref[...], ws_ref[...], (((1,), (1,)), ((), ())),
                       preferred_element_type=jnp.float32)
  hn = lax.dot_general(aggn, wn_ref[...], (((1,), (1,)), ((), ())),
                       preferred_element_type=jnp.float32)
  out_ref[...] = jnp.maximum(hs + hn + b_ref[...], 0.0)


def _layer_tc(h, agg, deg, ws, wn, b):
  n, d = h.shape
  grid = (n // BLK,)
  return pl.pallas_call(
      _layer_body,
      grid=grid,
      in_specs=[
          pl.BlockSpec((BLK, d), lambda i: (i, 0)),
          pl.BlockSpec((NC, BLK, d), lambda i: (0, i, 0)),
          pl.BlockSpec((NC, BLK, 1), lambda i: (0, i, 0)),
          pl.BlockSpec((d, d), lambda i: (0, 0)),
          pl.BlockSpec((d, d), lambda i: (0, 0)),
          pl.BlockSpec((1, d), lambda i: (0, 0)),
      ],
      out_specs=pl.BlockSpec((BLK, d), lambda i: (i, 0)),
      out_shape=jax.ShapeDtypeStruct((n, d), jnp.float32),
  )(h, agg, deg, ws, wn, b)


def _heads_body(h_ref, agg_ref, deg_ref, ws_ref, wn_ref, b_ref,
                wmu_ref, wsig_ref, weps_ref, bmu_ref, bsig_ref, beps_ref,
                gidx_ref, wv_ref, bv_ref,
                adv_ref, val_ref, gsum_ref, gcnt_ref):
  i = pl.program_id(0)
  ni = pl.num_programs(0)

  a = agg_ref[0] + agg_ref[1]
  dg = jnp.sum(deg_ref[...], axis=0)
  aggn = a / jnp.maximum(dg, 1.0)
  hs = lax.dot_general(h_ref[...], ws_ref[...], (((1,), (1,)), ((), ())),
                       preferred_element_type=jnp.float32)
  hn = lax.dot_general(aggn, wn_ref[...], (((1,), (1,)), ((), ())),
                       preferred_element_type=jnp.float32)
  h2 = jnp.maximum(hs + hn + b_ref[...], 0.0)

  weff = wmu_ref[...] + wsig_ref[...] * weps_ref[...]
  beff = bmu_ref[...] + bsig_ref[...] * beps_ref[...]
  adv_ref[...] = lax.dot_general(h2, weff, (((1,), (1,)), ((), ())),
                                 preferred_element_type=jnp.float32) + beff

  gi = gidx_ref[0]
  gids = lax.broadcasted_iota(jnp.int32, (64, BLK), 0)
  onehot = (gids == jnp.broadcast_to(gi, (64, BLK))).astype(jnp.float32)

  @pl.when(i == 0)
  def _():
    gsum_ref[...] = jnp.zeros_like(gsum_ref)
    gcnt_ref[...] = jnp.zeros_like(gcnt_ref)

  gsum_ref[...] += lax.dot_general(onehot, h2, (((1,), (0,)), ((), ())),
                                   preferred_element_type=jnp.float32)
  gcnt_ref[...] += jnp.sum(onehot, axis=1, keepdims=True)

  @pl.when(i == ni - 1)
  def _():
    gmean = gsum_ref[...] / jnp.maximum(gcnt_ref[...], 1.0)
    vv = jnp.sum(gmean * wv_ref[...], axis=1, keepdims=True)
    val_ref[...] = vv + bv_ref[0, 0]


def _heads_tc(h, agg, deg, ws, wn, b, wmu, wsig, weps, bmu, bsig, beps,
              gidx, wv, bv):
  n, d = h.shape
  out_dim = wmu.shape[0]
  g = 64
  grid = (n // BLK,)
  return pl.pallas_call(
      _heads_body,
      grid=grid,
      in_specs=[
          pl.BlockSpec((BLK, d), lambda i: (i, 0)),
          pl.BlockSpec((NC, BLK, d), lambda i: (0, i, 0)),
          pl.BlockSpec((NC, BLK, 1), lambda i: (0, i, 0)),
          pl.BlockSpec((d, d), lambda i: (0, 0)),
          pl.BlockSpec((d, d), lambda i: (0, 0)),
          pl.BlockSpec((1, d), lambda i: (0, 0)),
          pl.BlockSpec((out_dim, d), lambda i: (0, 0)),
          pl.BlockSpec((out_dim, d), lambda i: (0, 0)),
          pl.BlockSpec((out_dim, d), lambda i: (0, 0)),
          pl.BlockSpec((1, out_dim), lambda i: (0, 0)),
          pl.BlockSpec((1, out_dim), lambda i: (0, 0)),
          pl.BlockSpec((1, out_dim), lambda i: (0, 0)),
          pl.BlockSpec((1, 1, BLK), lambda i: (i, 0, 0)),
          pl.BlockSpec((1, d), lambda i: (0, 0)),
          pl.BlockSpec((1, 1), lambda i: (0, 0)),
      ],
      out_specs=[
          pl.BlockSpec((BLK, out_dim), lambda i: (i, 0)),
          pl.BlockSpec((g, 1), lambda i: (0, 0)),
      ],
      out_shape=[
          jax.ShapeDtypeStruct((n, out_dim), jnp.float32),
          jax.ShapeDtypeStruct((g, 1), jnp.float32),
      ],
      scratch_shapes=[
          pltpu.VMEM((g, d), jnp.float32),
          pltpu.VMEM((g, 1), jnp.float32),
      ],
  )(h, agg, deg, ws, wn, b, wmu, wsig, weps, bmu, bsig, beps, gidx, wv, bv)


def kernel(x, edge_index, graph_indices, W1s, W1n, b1, W2s, W2n, b2,
           w_mu, w_sigma, w_eps, b_mu, b_sigma, b_eps, Wv, bv):
  n, d = x.shape
  eidx = (edge_index[0].reshape(NW, ITERS, CH),
          edge_index[1].reshape(NW, ITERS, CH))

  agg1, deg = _seg_sum_sc(x, eidx, with_deg=True)
  deg3 = deg.reshape(NC, NP, 1)
  h1 = _layer_tc(x, agg1, deg3, W1s, W1n, b1.reshape(1, d))

  agg2 = _seg_sum_sc(h1, eidx, with_deg=False)

  adv, val = _heads_tc(
      h1, agg2, deg3, W2s, W2n, b2.reshape(1, d),
      w_mu, w_sigma, w_eps,
      b_mu.reshape(1, -1), b_sigma.reshape(1, -1), b_eps.reshape(1, -1),
      graph_indices.reshape(n // BLK, 1, BLK), Wv, bv.reshape(1, 1))
  return adv, val
